# speedup vs baseline: 15.7940x; 15.7940x over previous
import functools

import jax
import jax.numpy as jnp
from jax import lax
from jax.experimental import pallas as pl
from jax.experimental.pallas import tpu as pltpu
from jax.experimental.pallas import tpu_sc as plsc

B = 16384
L = 50
DIM = 128
NUM_CLASS = 1000

NC = 2
NS = 16
NW = NC * NS

GROUPS = B
G_PER_W = GROUPS // NW
NBUF = 4
FLUSH_T = 4
NT = G_PER_W // NBUF

BM = 512


def _sc_pool(idx2, table):
    mesh = plsc.VectorSubcoreMesh(core_axis_name="c", subcore_axis_name="s")

    @functools.partial(
        pl.kernel,
        out_type=jax.ShapeDtypeStruct((2 * B, DIM), jnp.float32),
        mesh=mesh,
        scratch_types=[
            pltpu.VMEM((G_PER_W, 2 * L), jnp.int32),
            pltpu.VMEM((NBUF, 2 * L, DIM), jnp.float32),
            pltpu.VMEM((2 * NBUF * FLUSH_T, DIM), jnp.float32),
            pltpu.SemaphoreType.DMA((NBUF,)),
        ],
    )
    def sc_pool(idx_hbm, table_hbm, out_hbm, idx_v, bufs, out_v, sems):
        wid = lax.axis_index("s") * NC + lax.axis_index("c")
        g0 = wid * G_PER_W
        row0 = wid * 2 * G_PER_W

        pltpu.sync_copy(idx_hbm.at[pl.ds(g0, G_PER_W)], idx_v)

        def fire(g, b):
            pltpu.make_async_copy(
                table_hbm.at[idx_v.at[g]], bufs.at[b], sems.at[b]
            ).start()

        def drain(g, b):
            pltpu.make_async_copy(
                table_hbm.at[idx_v.at[g]], bufs.at[b], sems.at[b]
            ).wait()

        for b in range(NBUF):
            fire(b, b)

        def outer(t, carry):
            for b in range(NBUF):
                g = t * NBUF + b
                drain(g, b)
                for j in range(2):
                    def body(r, accs):
                        row = j * L + 2 * r
                        return tuple(
                            accs[k]
                            + bufs[b, row, pl.ds(k * 16, 16)]
                            + bufs[b, row + 1, pl.ds(k * 16, 16)]
                            for k in range(8)
                        )

                    accs = lax.fori_loop(
                        0, L // 2, body,
                        tuple(jnp.zeros((16,), jnp.float32) for _ in range(8)),
                    )
                    lr = ((t % FLUSH_T) * NBUF + b) * 2 + j
                    for k in range(8):
                        out_v[lr, pl.ds(k * 16, 16)] = accs[k]

                @pl.when(t + 1 < NT)
                def _():
                    fire(g + NBUF, b)

            @pl.when(t % FLUSH_T == FLUSH_T - 1)
            def _():
                base = row0 + (t - (FLUSH_T - 1)) * 2 * NBUF
                pltpu.sync_copy(
                    out_v, out_hbm.at[pl.ds(base, 2 * NBUF * FLUSH_T)]
                )

            return carry

        lax.fori_loop(0, NT, outer, 0)

    return sc_pool(idx2, table)


def _mlp_body(pooled_ref, head_ref, tail_ref, lab_ref, W1_ref, b1_ref,
              W2_ref, b2_ref, logits_ref, loss_ref):
    i = pl.program_id(0)
    pooled = pooled_ref[...]
    hc = jnp.sum((head_ref[...] != 0).astype(jnp.int32), axis=1,
                 keepdims=True).astype(jnp.float32)
    tc = jnp.sum((tail_ref[...] != 0).astype(jnp.int32), axis=1,
                 keepdims=True).astype(jnp.float32)
    x1 = pooled[:, :DIM] / hc
    x2 = pooled[:, DIM:] / tc
    x = jnp.concatenate([x1, x2], axis=1)
    h = jnp.maximum(
        jnp.dot(x, W1_ref[...], preferred_element_type=jnp.float32)
        + b1_ref[...], 0.0)
    logits = (jnp.dot(h, W2_ref[...], preferred_element_type=jnp.float32)
              + b2_ref[...])
    logits_ref[...] = logits
    m = jnp.max(logits, axis=1, keepdims=True)
    lse = jnp.log(jnp.sum(jnp.exp(logits - m), axis=1, keepdims=True)) + m
    col = lax.broadcasted_iota(jnp.int32, logits.shape, 1)
    picked = jnp.sum(jnp.where(col == lab_ref[...], logits, 0.0), axis=1,
                     keepdims=True)
    nll = lse - picked

    @pl.when(i == 0)
    def _():
        loss_ref[...] = jnp.zeros_like(loss_ref)

    loss_ref[...] += (jnp.sum(nll) * (1.0 / B)).reshape(1, 1)


def _mlp(pooled, head, tail, labels2d, W1, b1, W2, b2):
    grid = (B // BM,)
    return pl.pallas_call(
        _mlp_body,
        grid=grid,
        in_specs=[
            pl.BlockSpec((BM, 2 * DIM), lambda i: (i, 0)),
            pl.BlockSpec((BM, L), lambda i: (i, 0)),
            pl.BlockSpec((BM, L), lambda i: (i, 0)),
            pl.BlockSpec((BM, 1), lambda i: (i, 0)),
            pl.BlockSpec((2 * DIM, DIM), lambda i: (0, 0)),
            pl.BlockSpec((1, DIM), lambda i: (0, 0)),
            pl.BlockSpec((DIM, NUM_CLASS), lambda i: (0, 0)),
            pl.BlockSpec((1, NUM_CLASS), lambda i: (0, 0)),
        ],
        out_specs=[
            pl.BlockSpec((BM, NUM_CLASS), lambda i: (i, 0)),
            pl.BlockSpec((1, 1), lambda i: (0, 0)),
        ],
        out_shape=[
            jax.ShapeDtypeStruct((B, NUM_CLASS), jnp.float32),
            jax.ShapeDtypeStruct((1, 1), jnp.float32),
        ],
    )(pooled, head, tail, labels2d, W1, b1, W2, b2)


def kernel(head, tail, labels, emb_table, W1, b1, W2, b2):
    head = head.astype(jnp.int32)
    tail = tail.astype(jnp.int32)
    idx2 = jnp.concatenate([head, tail], axis=1)
    pooled = _sc_pool(idx2, emb_table).reshape(B, 2 * DIM)
    logits, loss_acc = _mlp(pooled, head, tail, labels.reshape(B, 1),
                            W1, b1.reshape(1, DIM), W2,
                            b2.reshape(1, NUM_CLASS))
    return logits, loss_acc[0, 0]


# speedup vs baseline: 19.1411x; 1.2119x over previous
import functools

import jax
import jax.numpy as jnp
from jax import lax
from jax.experimental import pallas as pl
from jax.experimental.pallas import tpu as pltpu
from jax.experimental.pallas import tpu_sc as plsc

B = 16384
L = 50
DIM = 128
NUM_CLASS = 1000

NC = 2
NS = 16
NW = NC * NS

G_PER_W = B // NW
NBUF = 4
FLUSH_T = 4
NT = G_PER_W // NBUF

BM = 512


def _sc_pool(idx2, table):
    mesh = plsc.VectorSubcoreMesh(core_axis_name="c", subcore_axis_name="s")

    @functools.partial(
        pl.kernel,
        out_type=jax.ShapeDtypeStruct((B, 2 * DIM), jnp.float32),
        mesh=mesh,
        scratch_types=[
            pltpu.VMEM((G_PER_W, 2 * L), jnp.int32),
            pltpu.VMEM((NBUF, 2 * L, DIM), jnp.float32),
            pltpu.VMEM((NBUF * FLUSH_T, 2 * DIM), jnp.float32),
            pltpu.SemaphoreType.DMA((NBUF,)),
        ],
    )
    def sc_pool(idx_hbm, table_hbm, out_hbm, idx_v, bufs, out_v, sems):
        wid = lax.axis_index("s") * NC + lax.axis_index("c")
        g0 = wid * G_PER_W
        row0 = wid * G_PER_W

        pltpu.sync_copy(idx_hbm.at[pl.ds(g0, G_PER_W)], idx_v)

        def fire(g, b):
            pltpu.make_async_copy(
                table_hbm.at[idx_v.at[g]], bufs.at[b], sems.at[b]
            ).start()

        def drain(g, b):
            pltpu.make_async_copy(
                table_hbm.at[idx_v.at[g]], bufs.at[b], sems.at[b]
            ).wait()

        for b in range(NBUF):
            fire(b, b)

        def outer(t, carry):
            for b in range(NBUF):
                g = t * NBUF + b
                drain(g, b)
                lr = (t % FLUSH_T) * NBUF + b
                for j in range(2):
                    def body(r, accs):
                        row = j * L + 2 * r
                        return tuple(
                            accs[k]
                            + bufs[b, row, pl.ds(k * 16, 16)]
                            + bufs[b, row + 1, pl.ds(k * 16, 16)]
                            for k in range(8)
                        )

                    accs = lax.fori_loop(
                        0, L // 2, body,
                        tuple(jnp.zeros((16,), jnp.float32) for _ in range(8)),
                    )
                    for k in range(8):
                        out_v[lr, pl.ds(j * DIM + k * 16, 16)] = accs[k]

                @pl.when(t + 1 < NT)
                def _():
                    fire(g + NBUF, b)

            @pl.when(t % FLUSH_T == FLUSH_T - 1)
            def _():
                base = pl.multiple_of(
                    row0 + (t // FLUSH_T) * (NBUF * FLUSH_T), NBUF * FLUSH_T)
                pltpu.sync_copy(
                    out_v, out_hbm.at[pl.ds(base, NBUF * FLUSH_T)]
                )

            return carry

        lax.fori_loop(0, NT, outer, 0)

    return sc_pool(idx2, table)


def _mlp_body(pooled_ref, head_ref, tail_ref, lab_ref, W1_ref, b1_ref,
              W2_ref, b2T_ref, logitsT_ref, loss_ref):
    i = pl.program_id(0)
    pooled = pooled_ref[...]
    hc = jnp.sum((head_ref[...] != 0).astype(jnp.int32), axis=1,
                 keepdims=True).astype(jnp.float32)
    tc = jnp.sum((tail_ref[...] != 0).astype(jnp.int32), axis=1,
                 keepdims=True).astype(jnp.float32)
    x = jnp.concatenate([pooled[:, :DIM] / hc, pooled[:, DIM:] / tc],
                        axis=1)
    h = jnp.maximum(
        jnp.dot(x, W1_ref[...], preferred_element_type=jnp.float32)
        + b1_ref[...], 0.0)
    logitsT = lax.dot_general(
        W2_ref[...], h, (((0,), (1,)), ((), ())),
        preferred_element_type=jnp.float32) + b2T_ref[...]
    logitsT_ref[...] = logitsT
    m = jnp.max(logitsT, axis=0, keepdims=True)
    lse = jnp.log(jnp.sum(jnp.exp(logitsT - m), axis=0, keepdims=True)) + m
    cls = lax.broadcasted_iota(jnp.int32, logitsT.shape, 0)
    picked = jnp.sum(jnp.where(cls == lab_ref[...], logitsT, 0.0), axis=0,
                     keepdims=True)
    nll = lse - picked

    @pl.when(i == 0)
    def _():
        loss_ref[...] = jnp.zeros_like(loss_ref)

    loss_ref[...] += (jnp.sum(nll) * (1.0 / B)).reshape(1, 1)


def _mlp(pooled, head, tail, labels_row, W1, b1, W2, b2T):
    grid = (B // BM,)
    return pl.pallas_call(
        _mlp_body,
        grid=grid,
        in_specs=[
            pl.BlockSpec((BM, 2 * DIM), lambda i: (i, 0)),
            pl.BlockSpec((BM, L), lambda i: (i, 0)),
            pl.BlockSpec((BM, L), lambda i: (i, 0)),
            pl.BlockSpec((1, BM), lambda i: (0, i)),
            pl.BlockSpec((2 * DIM, DIM), lambda i: (0, 0)),
            pl.BlockSpec((1, DIM), lambda i: (0, 0)),
            pl.BlockSpec((DIM, NUM_CLASS), lambda i: (0, 0)),
            pl.BlockSpec((NUM_CLASS, 1), lambda i: (0, 0)),
        ],
        out_specs=[
            pl.BlockSpec((NUM_CLASS, BM), lambda i: (0, i)),
            pl.BlockSpec((1, 1), lambda i: (0, 0)),
        ],
        out_shape=[
            jax.ShapeDtypeStruct((NUM_CLASS, B), jnp.float32),
            jax.ShapeDtypeStruct((1, 1), jnp.float32),
        ],
    )(pooled, head, tail, labels_row, W1, b1, W2, b2T)


def kernel(head, tail, labels, emb_table, W1, b1, W2, b2):
    head = head.astype(jnp.int32)
    tail = tail.astype(jnp.int32)
    idx2 = jnp.concatenate([head, tail], axis=1)
    pooled = _sc_pool(idx2, emb_table)
    logitsT, loss_acc = _mlp(pooled, head, tail, labels.reshape(1, B), W1,
                             b1.reshape(1, DIM), W2,
                             b2.reshape(NUM_CLASS, 1))
    return logitsT.T, loss_acc[0, 0]
